# baseline (speedup 1.0000x reference)
import functools

import jax
import jax.numpy as jnp
from jax import lax
from jax.experimental import pallas as pl
from jax.experimental.pallas import tpu as pltpu
from jax.experimental.pallas import tpu_sc as plsc

N_NODES = 10000
N_PAD = 10240
N_EDGES = 160000
D = 128

NC = 2
NS = 16
NW = NC * NS
E_PER_TILE = N_EDGES // NW
CHUNK = 128
N_FULL = E_PER_TILE // CHUNK
TAIL = E_PER_TILE - N_FULL * CHUNK
N_CHUNKS = N_FULL + 1
TRASH = N_PAD - 1
ROWS_PER_TILE = N_PAD // NS

ACHUNK = 112
A_FULL = 44
A_TAIL = E_PER_TILE - A_FULL * ACHUNK
A_CHUNKS = A_FULL + 1
NBUF = 3
GD = 2


def _mesh():
    return plsc.VectorSubcoreMesh(core_axis_name="c", subcore_axis_name="s")


def _zero_fill(ref, rows, width):
    z = jnp.zeros((16,), jnp.float32)

    def body(i, _):
        for j in range(width // 16):
            ref[i, pl.ds(j * 16, 16)] = z
        return 0

    lax.fori_loop(0, rows, body, 0)


def _ones_fill(ref, rows, width):
    one = jnp.ones((16,), jnp.float32)

    def body(i, _):
        for j in range(width // 16):
            ref[i, pl.ds(j * 16, 16)] = one
        return 0

    lax.fori_loop(0, rows, body, 0)


@functools.lru_cache(maxsize=None)
def _deg_kernel():
    @functools.partial(
        pl.kernel,
        out_type=jax.ShapeDtypeStruct((NC, N_PAD, D), jnp.float32),
        mesh=_mesh(),
        scratch_types=[
            pltpu.VMEM((N_CHUNKS, CHUNK), jnp.int32),
            pltpu.VMEM((16,), jnp.int32),
            pltpu.VMEM((CHUNK, D), jnp.float32),
            pltpu.VMEM_SHARED((N_PAD, D), jnp.float32),
            pltpu.SemaphoreType.DMA,
            pltpu.SemaphoreType.DMA,
        ],
    )
    def deg(dst_hbm, out_hbm, didx2, didx_t, ones_v, acc, sem_i, sem_s):
        cid = lax.axis_index("c")
        sid = lax.axis_index("s")
        base = (cid * NS + sid) * E_PER_TILE

        trash = jnp.full((16,), TRASH, jnp.int32)
        didx_t[...] = trash
        for j in range(CHUNK // 16):
            didx2[N_FULL, pl.ds(j * 16, 16)] = trash

        idx_cps = [
            pltpu.async_copy(dst_hbm.at[pl.ds(base + j * CHUNK, CHUNK)],
                             didx2.at[j], sem_i)
            for j in range(N_FULL)
        ]
        idx_cps.append(
            pltpu.async_copy(dst_hbm.at[pl.ds(base + N_FULL * CHUNK, TAIL)],
                             didx_t.at[pl.ds(0, TAIL)], sem_i))

        _zero_fill(ones_v, CHUNK, D)
        r0 = sid * ROWS_PER_TILE
        for k in range(ROWS_PER_TILE // CHUNK):
            pltpu.sync_copy(ones_v, acc.at[pl.ds(r0 + k * CHUNK, CHUNK)])
        _ones_fill(ones_v, CHUNK, D)
        for cp in idx_cps:
            cp.wait()
        didx2[N_FULL, pl.ds(0, 16)] = didx_t[...]
        plsc.subcore_barrier()

        sc_cps = [
            pltpu.async_copy(ones_v, acc.at[didx2.at[j]], sem_s, add=True)
            for j in range(N_CHUNKS)
        ]
        for cp in sc_cps:
            cp.wait()
        plsc.subcore_barrier()

        pltpu.sync_copy(acc.at[pl.ds(r0, ROWS_PER_TILE)],
                        out_hbm.at[cid, pl.ds(r0, ROWS_PER_TILE)])

    return deg


@functools.lru_cache(maxsize=None)
def _agg_kernel():
    @functools.partial(
        pl.kernel,
        out_type=jax.ShapeDtypeStruct((NC, N_PAD, D), jnp.float32),
        mesh=_mesh(),
        scratch_types=[
            pltpu.VMEM((A_CHUNKS * ACHUNK,), jnp.int32),
            *[pltpu.VMEM((ACHUNK,), jnp.int32) for _ in range(NBUF)],
            *[pltpu.VMEM((ACHUNK, D), jnp.float32) for _ in range(NBUF)],
            pltpu.VMEM_SHARED((N_PAD, D), jnp.float32),
            *[pltpu.SemaphoreType.DMA for _ in range(3 * NBUF)],
            pltpu.SemaphoreType.DMA,
        ],
    )
    def agg(x2_hbm, src_hbm, dst_hbm, out_hbm, sidx, *rest):
        dring = list(rest[:NBUF])
        bufs = list(rest[NBUF:2 * NBUF])
        acc = rest[2 * NBUF]
        sem_g = list(rest[2 * NBUF + 1:3 * NBUF + 1])
        sem_s = list(rest[3 * NBUF + 1:4 * NBUF + 1])
        sem_d = list(rest[4 * NBUF + 1:5 * NBUF + 1])
        sem_i = rest[5 * NBUF + 1]
        cid = lax.axis_index("c")
        sid = lax.axis_index("s")
        base = (cid * NS + sid) * E_PER_TILE

        zero16 = jnp.zeros((16,), jnp.int32)
        trash = jnp.full((16,), TRASH, jnp.int32)
        pad0 = (E_PER_TILE // 16) * 16
        for o in range(pad0, A_CHUNKS * ACHUNK, 16):
            sidx[pl.ds(o, 16)] = zero16

        idx_cps = [
            pltpu.async_copy(src_hbm.at[pl.ds(base, A_FULL * ACHUNK)],
                             sidx.at[pl.ds(0, A_FULL * ACHUNK)], sem_i),
            pltpu.async_copy(src_hbm.at[pl.ds(base + A_FULL * ACHUNK, A_TAIL)],
                             sidx.at[pl.ds(A_FULL * ACHUNK, A_TAIL)], sem_i),
        ]

        _zero_fill(bufs[0], ACHUNK, D)
        r0 = sid * ROWS_PER_TILE
        nfullz = ROWS_PER_TILE // ACHUNK
        remz = ROWS_PER_TILE - nfullz * ACHUNK
        for k in range(nfullz):
            pltpu.sync_copy(bufs[0], acc.at[pl.ds(r0 + k * ACHUNK, ACHUNK)])
        pltpu.sync_copy(bufs[0].at[pl.ds(0, remz)],
                        acc.at[pl.ds(r0 + nfullz * ACHUNK, remz)])
        for cp in idx_cps:
            cp.wait()
        plsc.subcore_barrier()

        def gather(j, b):
            return pltpu.async_copy(
                x2_hbm.at[sidx.at[pl.ds(j * ACHUNK, ACHUNK)]], bufs[b],
                sem_g[b])

        def load_didx(j, b):
            return pltpu.async_copy(
                dst_hbm.at[pl.ds(base + j * ACHUNK, ACHUNK)], dring[b],
                sem_d[b])

        def scatter(b):
            return pltpu.async_copy(bufs[b], acc.at[dring[b]], sem_s[b],
                                    add=True)

        def wait_gather(b):
            pltpu.make_async_copy(x2_hbm.at[pl.ds(0, ACHUNK)], bufs[b],
                                  sem_g[b]).wait()

        def wait_didx(b):
            pltpu.make_async_copy(dst_hbm.at[pl.ds(0, ACHUNK)], dring[b],
                                  sem_d[b]).wait()

        def wait_scatter(b):
            pltpu.make_async_copy(x2_hbm.at[pl.ds(0, ACHUNK)], bufs[b],
                                  sem_s[b]).wait()

        load_didx(0, 0); gather(0, 0)
        load_didx(1, 1); gather(1, 1)
        wait_gather(0); wait_didx(0)
        scatter(0)
        load_didx(2, 2); gather(2, 2)
        wait_gather(1); wait_didx(1)
        scatter(1)
        wait_scatter(0)
        load_didx(3, 0); gather(3, 0)

        def body(k, _):
            j0 = 2 + 3 * k
            for t in range(3):
                j = j0 + t
                b = (2 + t) % 3
                bn = (4 + t) % 3
                wait_gather(b)
                wait_didx(b)
                scatter(b)
                wait_scatter(bn)
                load_didx(j + 2, bn)
                gather(j + 2, bn)
            return 0

        lax.fori_loop(0, 12, body, 0)

        for j in range(38, 45):
            b = j % 3
            wait_gather(b)
            if j == 44:
                pltpu.make_async_copy(dst_hbm.at[pl.ds(0, A_TAIL)],
                                      dring[b].at[pl.ds(0, A_TAIL)],
                                      sem_d[b]).wait()
            else:
                wait_didx(b)
            scatter(b)
            nj = j + 2
            if nj <= 44:
                bn = nj % 3
                wait_scatter(bn)
                if nj == 44:
                    for o in range(0, ACHUNK, 16):
                        dring[bn][pl.ds(o, 16)] = trash
                    pltpu.async_copy(
                        dst_hbm.at[pl.ds(base + nj * ACHUNK, A_TAIL)],
                        dring[bn].at[pl.ds(0, A_TAIL)], sem_d[bn])
                else:
                    load_didx(nj, bn)
                gather(nj, bn)
        for j in range(42, 45):
            wait_scatter(j % 3)
        plsc.subcore_barrier()

        pltpu.sync_copy(acc.at[pl.ds(r0, ROWS_PER_TILE)],
                        out_hbm.at[cid, pl.ds(r0, ROWS_PER_TILE)])

    return agg


BM = 1000


def _tc_body(ap_ref, dp_ref, x1_ref, w_ref, b_ref, out_ref):
    a = ap_ref[0] + ap_ref[1]
    d = dp_ref[0] + dp_ref[1]
    inv = 1.0 / jnp.maximum(d, 1.0)
    h = jnp.dot(a * inv, w_ref[...], preferred_element_type=jnp.float32)
    out_ref[...] = jnp.maximum(h + b_ref[...], 0.0) + x1_ref[...]


@functools.lru_cache(maxsize=None)
def _tc_kernel():
    grid = (N_NODES // BM,)
    return pl.pallas_call(
        _tc_body,
        grid=grid,
        in_specs=[
            pl.BlockSpec((NC, BM, D), lambda i: (0, i, 0)),
            pl.BlockSpec((NC, BM, 1), lambda i: (0, i, 0)),
            pl.BlockSpec((BM, D), lambda i: (i, 0)),
            pl.BlockSpec((D, D), lambda i: (0, 0)),
            pl.BlockSpec((1, D), lambda i: (0, 0)),
        ],
        out_specs=pl.BlockSpec((BM, D), lambda i: (i, 0)),
        out_shape=jax.ShapeDtypeStruct((N_NODES, D), jnp.float32),
    )


def kernel(x, edge_index, W1, b1, W2, b2, W3, b3):
    src = edge_index[0]
    dst = edge_index[1]
    deg_col = _deg_kernel()(dst)[:, :, :1]
    x1 = x[:, :D]
    x2 = x[:, D:]
    tc = _tc_kernel()
    for W, b in ((W1, b1), (W2, b2), (W3, b3)):
        agg_parts = _agg_kernel()(x2, src, dst)
        y2 = tc(agg_parts, deg_col, x1, W, b.reshape(1, D))
        x1, x2 = x2, y2
    out = jnp.concatenate([x1, x2], axis=1)
    return (out, out)


# speedup vs baseline: 1.4063x; 1.4063x over previous
import functools

import jax
import jax.numpy as jnp
from jax import lax
from jax.experimental import pallas as pl
from jax.experimental.pallas import tpu as pltpu
from jax.experimental.pallas import tpu_sc as plsc

N_NODES = 10000
N_PAD = 10240
N_EDGES = 160000
D = 128

NC = 2
NS = 16
NW = NC * NS
E_PER_TILE = N_EDGES // NW
CHUNK = 128
N_FULL = E_PER_TILE // CHUNK
TAIL = E_PER_TILE - N_FULL * CHUNK
N_CHUNKS = N_FULL + 1
TRASH = N_PAD - 1
ROWS_PER_TILE = N_PAD // NS

ACHUNK = 112
A_FULL = 44
A_TAIL = E_PER_TILE - A_FULL * ACHUNK
A_CHUNKS = A_FULL + 1
NBUF = 3
GD = 2


def _mesh():
    return plsc.VectorSubcoreMesh(core_axis_name="c", subcore_axis_name="s")


def _zero_fill(ref, rows, width):
    z = jnp.zeros((16,), jnp.float32)

    def body(i, _):
        for j in range(width // 16):
            ref[i, pl.ds(j * 16, 16)] = z
        return 0

    lax.fori_loop(0, rows, body, 0)


def _ones_fill(ref, rows, width):
    one = jnp.ones((16,), jnp.float32)

    def body(i, _):
        for j in range(width // 16):
            ref[i, pl.ds(j * 16, 16)] = one
        return 0

    lax.fori_loop(0, rows, body, 0)


@functools.lru_cache(maxsize=None)
def _deg_kernel():
    @functools.partial(
        pl.kernel,
        out_type=jax.ShapeDtypeStruct((NC, N_PAD, D), jnp.float32),
        mesh=_mesh(),
        scratch_types=[
            pltpu.VMEM((N_CHUNKS, CHUNK), jnp.int32),
            pltpu.VMEM((16,), jnp.int32),
            pltpu.VMEM((CHUNK, D), jnp.float32),
            pltpu.VMEM_SHARED((N_PAD, D), jnp.float32),
            pltpu.SemaphoreType.DMA,
            pltpu.SemaphoreType.DMA,
        ],
    )
    def deg(dst_hbm, out_hbm, didx2, didx_t, ones_v, acc, sem_i, sem_s):
        cid = lax.axis_index("c")
        sid = lax.axis_index("s")
        base = (cid * NS + sid) * E_PER_TILE

        trash = jnp.full((16,), TRASH, jnp.int32)
        didx_t[...] = trash
        for j in range(CHUNK // 16):
            didx2[N_FULL, pl.ds(j * 16, 16)] = trash

        idx_cps = [
            pltpu.async_copy(dst_hbm.at[pl.ds(base + j * CHUNK, CHUNK)],
                             didx2.at[j], sem_i)
            for j in range(N_FULL)
        ]
        idx_cps.append(
            pltpu.async_copy(dst_hbm.at[pl.ds(base + N_FULL * CHUNK, TAIL)],
                             didx_t.at[pl.ds(0, TAIL)], sem_i))

        _zero_fill(ones_v, CHUNK, D)
        r0 = sid * ROWS_PER_TILE
        for k in range(ROWS_PER_TILE // CHUNK):
            pltpu.sync_copy(ones_v, acc.at[pl.ds(r0 + k * CHUNK, CHUNK)])
        _ones_fill(ones_v, CHUNK, D)
        for cp in idx_cps:
            cp.wait()
        didx2[N_FULL, pl.ds(0, 16)] = didx_t[...]
        plsc.subcore_barrier()

        sc_cps = [
            pltpu.async_copy(ones_v, acc.at[didx2.at[j]], sem_s, add=True)
            for j in range(N_CHUNKS)
        ]
        for cp in sc_cps:
            cp.wait()
        plsc.subcore_barrier()

        pltpu.sync_copy(acc.at[pl.ds(r0, ROWS_PER_TILE)],
                        out_hbm.at[cid, pl.ds(r0, ROWS_PER_TILE)])

    return deg


@functools.lru_cache(maxsize=None)
def _agg_kernel():
    @functools.partial(
        pl.kernel,
        out_type=jax.ShapeDtypeStruct((NC, N_PAD, D), jnp.float32),
        mesh=_mesh(),
        scratch_types=[
            pltpu.VMEM((N_FULL * CHUNK,), jnp.int32),
            pltpu.VMEM((N_FULL, CHUNK), jnp.int32),
            pltpu.VMEM((TAIL,), jnp.int32),
            pltpu.VMEM((TAIL,), jnp.int32),
            pltpu.VMEM((CHUNK, D), jnp.float32),
            pltpu.VMEM((CHUNK, D), jnp.float32),
            pltpu.VMEM((TAIL, D), jnp.float32),
            pltpu.VMEM_SHARED((N_PAD, D), jnp.float32),
            pltpu.SemaphoreType.DMA,
            pltpu.SemaphoreType.DMA,
            pltpu.SemaphoreType.DMA,
        ],
    )
    def agg(x2_hbm, src_hbm, dst_hbm, out_hbm,
            sidx, didx2, sidx_t, didx_t, rows_a, rows_b, rows_t,
            acc, sem_a, sem_b, sem_i):
        cid = lax.axis_index("c")
        sid = lax.axis_index("s")
        base = (cid * NS + sid) * E_PER_TILE

        idx_cps = [
            pltpu.async_copy(src_hbm.at[pl.ds(base, N_FULL * CHUNK)], sidx,
                             sem_i),
            pltpu.async_copy(src_hbm.at[pl.ds(base + N_FULL * CHUNK, TAIL)],
                             sidx_t, sem_i),
            pltpu.async_copy(dst_hbm.at[pl.ds(base + N_FULL * CHUNK, TAIL)],
                             didx_t, sem_i),
        ]
        idx_cps += [
            pltpu.async_copy(dst_hbm.at[pl.ds(base + j * CHUNK, CHUNK)],
                             didx2.at[j], sem_i)
            for j in range(N_FULL)
        ]

        _zero_fill(rows_a, CHUNK, D)
        r0 = sid * ROWS_PER_TILE
        for k in range(ROWS_PER_TILE // CHUNK):
            pltpu.sync_copy(rows_a, acc.at[pl.ds(r0 + k * CHUNK, CHUNK)])
        for cp in idx_cps:
            cp.wait()
        plsc.subcore_barrier()

        def gather(j, buf, sem):
            return pltpu.async_copy(
                x2_hbm.at[sidx.at[pl.ds(j * CHUNK, CHUNK)]], buf, sem)

        gather(0, rows_a, sem_a).wait()

        def body(k, _):
            j0 = 2 * k
            j1 = j0 + 1
            gb = gather(j1, rows_b, sem_b)
            pltpu.sync_copy(rows_a, acc.at[didx2.at[j0]], add=True)
            ga = gather(j0 + 2, rows_a, sem_a)
            gb.wait()
            pltpu.sync_copy(rows_b, acc.at[didx2.at[j1]], add=True)
            ga.wait()
            return 0

        lax.fori_loop(0, (N_FULL - 1) // 2, body, 0)
        pltpu.async_copy(x2_hbm.at[sidx_t], rows_t, sem_b).wait()
        pltpu.sync_copy(rows_a, acc.at[didx2.at[N_FULL - 1]], add=True)
        pltpu.sync_copy(rows_t, acc.at[didx_t], add=True)
        plsc.subcore_barrier()

        pltpu.sync_copy(acc.at[pl.ds(r0, ROWS_PER_TILE)],
                        out_hbm.at[cid, pl.ds(r0, ROWS_PER_TILE)])

    return agg


BM = 1000


def _tc_body(ap_ref, dp_ref, x1_ref, w_ref, b_ref, out_ref):
    a = ap_ref[0] + ap_ref[1]
    d = dp_ref[0] + dp_ref[1]
    inv = 1.0 / jnp.maximum(d, 1.0)
    h = jnp.dot(a * inv, w_ref[...], preferred_element_type=jnp.float32)
    out_ref[...] = jnp.maximum(h + b_ref[...], 0.0) + x1_ref[...]


def _tc_last_body(ap_ref, dp_ref, x1_ref, xc_ref, w_ref, b_ref, out_ref):
    a = ap_ref[0] + ap_ref[1]
    d = dp_ref[0] + dp_ref[1]
    inv = 1.0 / jnp.maximum(d, 1.0)
    h = jnp.dot(a * inv, w_ref[...], preferred_element_type=jnp.float32)
    y = jnp.maximum(h + b_ref[...], 0.0) + x1_ref[...]
    out_ref[:, pl.ds(0, D)] = xc_ref[...]
    out_ref[:, pl.ds(D, D)] = y


@functools.lru_cache(maxsize=None)
def _tc_kernel():
    grid = (N_NODES // BM,)
    return pl.pallas_call(
        _tc_body,
        grid=grid,
        in_specs=[
            pl.BlockSpec((NC, BM, D), lambda i: (0, i, 0)),
            pl.BlockSpec((NC, BM, 1), lambda i: (0, i, 0)),
            pl.BlockSpec((BM, D), lambda i: (i, 0)),
            pl.BlockSpec((D, D), lambda i: (0, 0)),
            pl.BlockSpec((1, D), lambda i: (0, 0)),
        ],
        out_specs=pl.BlockSpec((BM, D), lambda i: (i, 0)),
        out_shape=jax.ShapeDtypeStruct((N_NODES, D), jnp.float32),
    )


@functools.lru_cache(maxsize=None)
def _tc_last_kernel():
    grid = (N_NODES // BM,)
    return pl.pallas_call(
        _tc_last_body,
        grid=grid,
        in_specs=[
            pl.BlockSpec((NC, BM, D), lambda i: (0, i, 0)),
            pl.BlockSpec((NC, BM, 1), lambda i: (0, i, 0)),
            pl.BlockSpec((BM, D), lambda i: (i, 0)),
            pl.BlockSpec((BM, D), lambda i: (i, 0)),
            pl.BlockSpec((D, D), lambda i: (0, 0)),
            pl.BlockSpec((1, D), lambda i: (0, 0)),
        ],
        out_specs=pl.BlockSpec((BM, 2 * D), lambda i: (i, 0)),
        out_shape=jax.ShapeDtypeStruct((N_NODES, 2 * D), jnp.float32),
    )


def kernel(x, edge_index, W1, b1, W2, b2, W3, b3):
    src = edge_index[0]
    dst = edge_index[1]
    deg_col = _deg_kernel()(dst)[:, :, :1]
    x2_0 = x[:, D:]
    tc = _tc_kernel()
    agg = _agg_kernel()
    y1 = tc(agg(x2_0, src, dst), deg_col, x, W1, b1.reshape(1, D))
    y2 = tc(agg(y1, src, dst), deg_col, x2_0, W2, b2.reshape(1, D))
    out = _tc_last_kernel()(agg(y2, src, dst), deg_col, y1, y2,
                            W3, b3.reshape(1, D))
    return (out, out)


# speedup vs baseline: 1.4137x; 1.0053x over previous
import functools

import jax
import jax.numpy as jnp
from jax import lax
from jax.experimental import pallas as pl
from jax.experimental.pallas import tpu as pltpu
from jax.experimental.pallas import tpu_sc as plsc

N_NODES = 10000
N_PAD = 10240
N_EDGES = 160000
D = 128

NC = 2
NS = 16
NW = NC * NS
E_PER_TILE = N_EDGES // NW
CHUNK = 128
N_FULL = E_PER_TILE // CHUNK
TAIL = E_PER_TILE - N_FULL * CHUNK
N_CHUNKS = N_FULL + 1
TRASH = N_PAD - 1
ROWS_PER_TILE = N_PAD // NS

ACHUNK = 112
A_FULL = 44
A_TAIL = E_PER_TILE - A_FULL * ACHUNK
A_CHUNKS = A_FULL + 1
NBUF = 3
GD = 2


def _mesh():
    return plsc.VectorSubcoreMesh(core_axis_name="c", subcore_axis_name="s")


def _zero_fill(ref, rows, width):
    z = jnp.zeros((16,), jnp.float32)

    def body(i, _):
        for j in range(width // 16):
            ref[i, pl.ds(j * 16, 16)] = z
        return 0

    lax.fori_loop(0, rows, body, 0)


def _ones_fill(ref, rows, width):
    one = jnp.ones((16,), jnp.float32)

    def body(i, _):
        for j in range(width // 16):
            ref[i, pl.ds(j * 16, 16)] = one
        return 0

    lax.fori_loop(0, rows, body, 0)


@functools.lru_cache(maxsize=None)
def _deg_kernel():
    @functools.partial(
        pl.kernel,
        out_type=jax.ShapeDtypeStruct((NC, N_PAD, D), jnp.float32),
        mesh=_mesh(),
        scratch_types=[
            pltpu.VMEM((N_CHUNKS, CHUNK), jnp.int32),
            pltpu.VMEM((16,), jnp.int32),
            pltpu.VMEM((CHUNK, D), jnp.float32),
            pltpu.VMEM_SHARED((N_PAD, D), jnp.float32),
            pltpu.SemaphoreType.DMA,
            pltpu.SemaphoreType.DMA,
        ],
    )
    def deg(dst_hbm, out_hbm, didx2, didx_t, ones_v, acc, sem_i, sem_s):
        cid = lax.axis_index("c")
        sid = lax.axis_index("s")
        base = (cid * NS + sid) * E_PER_TILE

        trash = jnp.full((16,), TRASH, jnp.int32)
        didx_t[...] = trash
        for j in range(CHUNK // 16):
            didx2[N_FULL, pl.ds(j * 16, 16)] = trash

        idx_cps = [
            pltpu.async_copy(dst_hbm.at[pl.ds(base + j * CHUNK, CHUNK)],
                             didx2.at[j], sem_i)
            for j in range(N_FULL)
        ]
        idx_cps.append(
            pltpu.async_copy(dst_hbm.at[pl.ds(base + N_FULL * CHUNK, TAIL)],
                             didx_t.at[pl.ds(0, TAIL)], sem_i))

        _zero_fill(ones_v, CHUNK, D)
        r0 = sid * ROWS_PER_TILE
        z_cps = [
            pltpu.async_copy(ones_v, acc.at[pl.ds(r0 + k * CHUNK, CHUNK)],
                             sem_s)
            for k in range(ROWS_PER_TILE // CHUNK)
        ]
        for cp in z_cps:
            cp.wait()
        _ones_fill(ones_v, CHUNK, D)
        for cp in idx_cps:
            cp.wait()
        didx2[N_FULL, pl.ds(0, 16)] = didx_t[...]
        plsc.subcore_barrier()

        sc_cps = [
            pltpu.async_copy(ones_v, acc.at[didx2.at[j]], sem_s, add=True)
            for j in range(N_CHUNKS)
        ]
        for cp in sc_cps:
            cp.wait()
        plsc.subcore_barrier()

        pltpu.sync_copy(acc.at[pl.ds(r0, ROWS_PER_TILE)],
                        out_hbm.at[cid, pl.ds(r0, ROWS_PER_TILE)])

    return deg


@functools.lru_cache(maxsize=None)
def _agg_kernel():
    @functools.partial(
        pl.kernel,
        out_type=jax.ShapeDtypeStruct((NC, N_PAD, D), jnp.float32),
        mesh=_mesh(),
        scratch_types=[
            pltpu.VMEM((N_FULL * CHUNK,), jnp.int32),
            pltpu.VMEM((N_FULL, CHUNK), jnp.int32),
            pltpu.VMEM((TAIL,), jnp.int32),
            pltpu.VMEM((TAIL,), jnp.int32),
            pltpu.VMEM((CHUNK, D), jnp.float32),
            pltpu.VMEM((CHUNK, D), jnp.float32),
            pltpu.VMEM((TAIL, D), jnp.float32),
            pltpu.VMEM_SHARED((N_PAD, D), jnp.float32),
            pltpu.SemaphoreType.DMA,
            pltpu.SemaphoreType.DMA,
            pltpu.SemaphoreType.DMA,
        ],
    )
    def agg(x2_hbm, src_hbm, dst_hbm, out_hbm,
            sidx, didx2, sidx_t, didx_t, rows_a, rows_b, rows_t,
            acc, sem_a, sem_b, sem_i):
        cid = lax.axis_index("c")
        sid = lax.axis_index("s")
        base = (cid * NS + sid) * E_PER_TILE

        idx_cps = [
            pltpu.async_copy(src_hbm.at[pl.ds(base, N_FULL * CHUNK)], sidx,
                             sem_i),
            pltpu.async_copy(src_hbm.at[pl.ds(base + N_FULL * CHUNK, TAIL)],
                             sidx_t, sem_i),
            pltpu.async_copy(dst_hbm.at[pl.ds(base + N_FULL * CHUNK, TAIL)],
                             didx_t, sem_i),
        ]
        idx_cps += [
            pltpu.async_copy(dst_hbm.at[pl.ds(base + j * CHUNK, CHUNK)],
                             didx2.at[j], sem_i)
            for j in range(N_FULL)
        ]

        _zero_fill(rows_a, CHUNK, D)
        r0 = sid * ROWS_PER_TILE
        z_cps = [
            pltpu.async_copy(rows_a, acc.at[pl.ds(r0 + k * CHUNK, CHUNK)],
                             sem_a)
            for k in range(ROWS_PER_TILE // CHUNK)
        ]
        for cp in z_cps:
            cp.wait()
        for cp in idx_cps:
            cp.wait()
        plsc.subcore_barrier()

        def gather(j, buf, sem):
            return pltpu.async_copy(
                x2_hbm.at[sidx.at[pl.ds(j * CHUNK, CHUNK)]], buf, sem)

        gather(0, rows_a, sem_a).wait()

        def body(k, _):
            j0 = 2 * k
            j1 = j0 + 1
            gb = gather(j1, rows_b, sem_b)
            pltpu.sync_copy(rows_a, acc.at[didx2.at[j0]], add=True)
            ga = gather(j0 + 2, rows_a, sem_a)
            gb.wait()
            pltpu.sync_copy(rows_b, acc.at[didx2.at[j1]], add=True)
            ga.wait()
            return 0

        lax.fori_loop(0, (N_FULL - 1) // 2, body, 0)
        pltpu.async_copy(x2_hbm.at[sidx_t], rows_t, sem_b).wait()
        pltpu.sync_copy(rows_a, acc.at[didx2.at[N_FULL - 1]], add=True)
        pltpu.sync_copy(rows_t, acc.at[didx_t], add=True)
        plsc.subcore_barrier()

        pltpu.sync_copy(acc.at[pl.ds(r0, ROWS_PER_TILE)],
                        out_hbm.at[cid, pl.ds(r0, ROWS_PER_TILE)])

    return agg


BM = 1000


def _tc_body(ap_ref, dp_ref, x1_ref, w_ref, b_ref, out_ref):
    a = ap_ref[0] + ap_ref[1]
    d = dp_ref[0] + dp_ref[1]
    inv = 1.0 / jnp.maximum(d, 1.0)
    h = jnp.dot(a * inv, w_ref[...], preferred_element_type=jnp.float32)
    out_ref[...] = jnp.maximum(h + b_ref[...], 0.0) + x1_ref[...]


def _tc_last_body(ap_ref, dp_ref, x1_ref, xc_ref, w_ref, b_ref, out_ref):
    a = ap_ref[0] + ap_ref[1]
    d = dp_ref[0] + dp_ref[1]
    inv = 1.0 / jnp.maximum(d, 1.0)
    h = jnp.dot(a * inv, w_ref[...], preferred_element_type=jnp.float32)
    y = jnp.maximum(h + b_ref[...], 0.0) + x1_ref[...]
    out_ref[:, pl.ds(0, D)] = xc_ref[...]
    out_ref[:, pl.ds(D, D)] = y


@functools.lru_cache(maxsize=None)
def _tc_kernel():
    grid = (N_NODES // BM,)
    return pl.pallas_call(
        _tc_body,
        grid=grid,
        in_specs=[
            pl.BlockSpec((NC, BM, D), lambda i: (0, i, 0)),
            pl.BlockSpec((NC, BM, 1), lambda i: (0, i, 0)),
            pl.BlockSpec((BM, D), lambda i: (i, 0)),
            pl.BlockSpec((D, D), lambda i: (0, 0)),
            pl.BlockSpec((1, D), lambda i: (0, 0)),
        ],
        out_specs=pl.BlockSpec((BM, D), lambda i: (i, 0)),
        out_shape=jax.ShapeDtypeStruct((N_NODES, D), jnp.float32),
    )


@functools.lru_cache(maxsize=None)
def _tc_last_kernel():
    grid = (N_NODES // BM,)
    return pl.pallas_call(
        _tc_last_body,
        grid=grid,
        in_specs=[
            pl.BlockSpec((NC, BM, D), lambda i: (0, i, 0)),
            pl.BlockSpec((NC, BM, 1), lambda i: (0, i, 0)),
            pl.BlockSpec((BM, D), lambda i: (i, 0)),
            pl.BlockSpec((BM, D), lambda i: (i, 0)),
            pl.BlockSpec((D, D), lambda i: (0, 0)),
            pl.BlockSpec((1, D), lambda i: (0, 0)),
        ],
        out_specs=pl.BlockSpec((BM, 2 * D), lambda i: (i, 0)),
        out_shape=jax.ShapeDtypeStruct((N_NODES, 2 * D), jnp.float32),
    )


def kernel(x, edge_index, W1, b1, W2, b2, W3, b3):
    src = edge_index[0]
    dst = edge_index[1]
    deg_col = _deg_kernel()(dst)[:, :, :1]
    x2_0 = x[:, D:]
    tc = _tc_kernel()
    agg = _agg_kernel()
    y1 = tc(agg(x2_0, src, dst), deg_col, x, W1, b1.reshape(1, D))
    y2 = tc(agg(y1, src, dst), deg_col, x2_0, W2, b2.reshape(1, D))
    out = _tc_last_kernel()(agg(y2, src, dst), deg_col, y1, y2,
                            W3, b3.reshape(1, D))
    return (out, out)
